# Initial kernel scaffold; baseline (speedup 1.0000x reference)
import jax
import jax.numpy as jnp
from jax.experimental import pallas as pl


def kernel(x, edge_index, batch_index, mlp1_W1, mlp1_b1, bn1_g, bn1_b, mlp1_W2, mlp1_b2, mlp2_W1, mlp2_b1, bn2_g, bn2_b, mlp2_W2, mlp2_b2, eps1, eps2, eps3, out1_W, out1_b, out2_W, out2_b, out3_W, out3_b, out_W, out_b):
    raise NotImplementedError("write your pallas kernel here")


# speedup vs baseline: 1.5627x; 1.5627x over previous
import functools

import jax
import jax.numpy as jnp
from jax import lax
from jax.experimental import pallas as pl
from jax.experimental.pallas import tpu as pltpu
from jax.experimental.pallas import tpu_sc as plsc

N = 50000
E = 800000
B = 128
H = 400
C = 32
NB1 = 3
NB2 = 13
F1P = NB1 * C
HP = NB2 * C
NCORES = 2
NTILES = 16
EROWS = 6400
RPT = EROWS // NTILES
CH = 4
OUTER = RPT // CH
NPT = N // NTILES
RB = 400
NRB = N // RB
G_POOL = NRB + B - 1


def _sc_aggregate(tab_flat, srcoff, dst2d, zrows, nb, cols):
    nfb_max = (nb + 1) // 2
    mesh = plsc.VectorSubcoreMesh(
        core_axis_name="c", subcore_axis_name="s",
        num_cores=NCORES, num_subcores=NTILES)

    def body(tab_hbm, src_hbm, dst_hbm, z_hbm, out_hbm,
             sbuf, dbuf, rows, acc, sem):
        c = lax.axis_index("c")
        s = lax.axis_index("s")
        for i in range(nfb_max):
            fb = i * 2 + c

            @pl.when(fb < nb)
            def _():
                pltpu.sync_copy(z_hbm, acc.at[pl.ds(s * NPT, NPT)])

                @pl.when(s == 0)
                def _():
                    pltpu.sync_copy(z_hbm.at[pl.ds(0, 8)], acc.at[pl.ds(N, 8)])
                plsc.subcore_barrier()

                @pl.loop(0, OUTER)
                def _(o):
                    row0 = s * RPT + o * CH
                    pltpu.sync_copy(
                        src_hbm.at[pl.ds(fb * EROWS + row0, CH)], sbuf)
                    pltpu.sync_copy(dst_hbm.at[pl.ds(row0, CH)], dbuf)
                    descs = []
                    for j in range(CH):
                        descs.append(pltpu.async_copy(
                            tab_hbm.at[sbuf.at[j]],
                            rows.at[pl.ds(j * 128, 128)], sem))
                    for d in descs:
                        d.wait()
                    for j in range(CH):
                        pltpu.sync_copy(rows.at[pl.ds(j * 128, 128)],
                                        acc.at[dbuf.at[j]], add=True)
                plsc.subcore_barrier()
                pltpu.sync_copy(
                    acc.at[pl.ds(s * NPT, NPT)],
                    out_hbm.at[pl.ds(s * NPT, NPT), pl.ds(fb * C, C)])
                plsc.subcore_barrier()

    return pl.kernel(
        body,
        out_type=jax.ShapeDtypeStruct((N, cols), jnp.float32),
        mesh=mesh,
        compiler_params=pltpu.CompilerParams(use_tc_tiling_on_sc=False),
        scratch_types=[
            pltpu.VMEM((CH, 128), jnp.int32),
            pltpu.VMEM((CH, 128), jnp.int32),
            pltpu.VMEM((CH * 128, C), jnp.float32),
            pltpu.VMEM_SHARED((N + 8, C), jnp.float32),
            pltpu.SemaphoreType.DMA,
        ],
    )(tab_flat, srcoff, dst2d, zrows)


def _mm1(prev2d, agg2d, w1, eps, fp):
    def body(eps_ref, prev_ref, agg_ref, w_ref, z_ref, st_ref):
        i = pl.program_id(0)
        e = eps_ref[0, 0]
        u = prev_ref[...] * (1.0 + e) + agg_ref[...]
        z = jnp.dot(u, w_ref[...], preferred_element_type=jnp.float32)
        z_ref[...] = z
        su = jnp.sum(z, axis=0)
        sq = jnp.sum(z * z, axis=0)
        upd = jnp.concatenate(
            [su[None], sq[None], jnp.zeros((6, H), jnp.float32)], axis=0)

        @pl.when(i == 0)
        def _():
            st_ref[...] = upd

        @pl.when(i > 0)
        def _():
            st_ref[...] = st_ref[...] + upd

    return pl.pallas_call(
        body,
        grid=(NRB,),
        in_specs=[
            pl.BlockSpec(memory_space=pltpu.SMEM),
            pl.BlockSpec((RB, fp), lambda i: (i, 0)),
            pl.BlockSpec((RB, fp), lambda i: (i, 0)),
            pl.BlockSpec((fp, H), lambda i: (0, 0)),
        ],
        out_specs=[
            pl.BlockSpec((RB, H), lambda i: (i, 0)),
            pl.BlockSpec((8, H), lambda i: (0, 0)),
        ],
        out_shape=[
            jax.ShapeDtypeStruct((N, H), jnp.float32),
            jax.ShapeDtypeStruct((8, H), jnp.float32),
        ],
    )(eps.reshape(1, 1), prev2d, agg2d, w1)


def _tail(z, st, consts, w2, ow, with_tab):
    def body(z_ref, st_ref, c_ref, w2_ref, ow_ref, t2d_ref, *tab):
        mu = st_ref[0:1, :] * (1.0 / N)
        var = st_ref[1:2, :] * (1.0 / N) - mu * mu
        inv = lax.rsqrt(var + 1e-5)
        scale = c_ref[0:1, :] * inv
        shift = c_ref[1:2, :] - mu * scale
        a = jnp.maximum(z_ref[...] * scale + shift, 0.0)
        h = jnp.maximum(
            jnp.dot(a, w2_ref[...], preferred_element_type=jnp.float32)
            + c_ref[2:3, :], 0.0)
        t = jnp.tanh(
            jnp.dot(h, ow_ref[...], preferred_element_type=jnp.float32)
            + c_ref[3:4, :])
        tp = jnp.concatenate([t, jnp.zeros((RB, HP - H), jnp.float32)], axis=1)
        t2d_ref[...] = tp
        if with_tab:
            for fb in range(NB2):
                tab[0][fb] = tp[:, fb * C:(fb + 1) * C]

    out_shape = [jax.ShapeDtypeStruct((N, HP), jnp.float32)]
    out_specs = [pl.BlockSpec((RB, HP), lambda i: (i, 0))]
    if with_tab:
        out_shape.append(jax.ShapeDtypeStruct((NB2, N, C), jnp.float32))
        out_specs.append(pl.BlockSpec((NB2, RB, C), lambda i: (0, i, 0)))
    res = pl.pallas_call(
        body,
        grid=(NRB,),
        in_specs=[
            pl.BlockSpec((RB, H), lambda i: (i, 0)),
            pl.BlockSpec((8, H), lambda i: (0, 0)),
            pl.BlockSpec((8, H), lambda i: (0, 0)),
            pl.BlockSpec((H, H), lambda i: (0, 0)),
            pl.BlockSpec((H, H), lambda i: (0, 0)),
        ],
        out_specs=out_specs,
        out_shape=out_shape,
    )(z, st, consts, w2, ow)
    return res if with_tab else res[0]


def _pool(t3, bid8, sched):
    def body(sched_ref, t_ref, bid_ref, mx_ref, sm_ref, ct_ref):
        g = pl.program_id(0)
        bat = sched_ref[1, g]
        first = sched_ref[2, g]
        valid = sched_ref[3, g]
        mf = (bid_ref[...][:, 0:1] == bat) & (valid > 0)
        tf = t_ref[:, :H]
        mx = jnp.max(jnp.where(mf, tf, -jnp.inf), axis=0, keepdims=True)
        sm = jnp.sum(jnp.where(mf, tf, 0.0), axis=0, keepdims=True)
        ct = jnp.sum(mf.astype(jnp.float32))

        @pl.when(first > 0)
        def _():
            mx_ref[...] = jnp.full((1, 1, H), -jnp.inf, jnp.float32)
            sm_ref[...] = jnp.zeros((1, 1, H), jnp.float32)
            ct_ref[...] = jnp.zeros((1, 1, H), jnp.float32)

        mx_ref[...] = jnp.maximum(mx_ref[...], mx[None])
        sm_ref[...] = sm_ref[...] + sm[None]
        ct_ref[...] = ct_ref[...] + ct

    grid_spec = pltpu.PrefetchScalarGridSpec(
        num_scalar_prefetch=1,
        grid=(G_POOL,),
        in_specs=[
            pl.BlockSpec((RB, HP), lambda g, sr: (sr[0, g], 0)),
            pl.BlockSpec((RB, 8), lambda g, sr: (sr[0, g], 0)),
        ],
        out_specs=[
            pl.BlockSpec((1, 1, H), lambda g, sr: (sr[1, g], 0, 0)),
            pl.BlockSpec((1, 1, H), lambda g, sr: (sr[1, g], 0, 0)),
            pl.BlockSpec((1, 1, H), lambda g, sr: (sr[1, g], 0, 0)),
        ],
    )
    return pl.pallas_call(
        body,
        grid_spec=grid_spec,
        out_shape=[
            jax.ShapeDtypeStruct((B, 1, H), jnp.float32),
            jax.ShapeDtypeStruct((B, 1, H), jnp.float32),
            jax.ShapeDtypeStruct((B, 1, H), jnp.float32),
        ],
    )(sched, t3, bid8)


def _final(mx, sm, ct, fconsts):
    def body(mx_ref, sm_ref, ct_ref, c_ref, o_ref):
        gmp = mx_ref[:, 0, :]
        gap = sm_ref[:, 0, :] / jnp.maximum(ct_ref[:, 0, :], 1.0)
        r = (jnp.sum(gmp * c_ref[0:1, :] + gap * c_ref[1:2, :],
                     axis=1, keepdims=True) + c_ref[2, 0])
        o_ref[...] = jnp.broadcast_to(r, (B, B))

    return pl.pallas_call(
        body,
        grid=(1,),
        in_specs=[
            pl.BlockSpec((B, 1, H), lambda i: (0, 0, 0)),
            pl.BlockSpec((B, 1, H), lambda i: (0, 0, 0)),
            pl.BlockSpec((B, 1, H), lambda i: (0, 0, 0)),
            pl.BlockSpec((8, H), lambda i: (0, 0)),
        ],
        out_specs=pl.BlockSpec((B, B), lambda i: (0, 0)),
        out_shape=jax.ShapeDtypeStruct((B, B), jnp.float32),
    )(mx, sm, ct, fconsts)


def _schedule(batch_index):
    bi = batch_index.astype(jnp.int32)
    starts = jnp.searchsorted(bi, jnp.arange(B + 1, dtype=jnp.int32))
    starts = starts.astype(jnp.int32)
    s_b = starts[:B]
    e_b = starts[1:]
    blk0 = jnp.clip(s_b // RB, 0, NRB - 1)
    blk1 = jnp.where(e_b > s_b, jnp.clip((e_b - 1) // RB, 0, NRB - 1), blk0)
    nblk = blk1 - blk0 + 1
    offs = jnp.concatenate(
        [jnp.zeros((1,), jnp.int32), jnp.cumsum(nblk).astype(jnp.int32)])
    total = offs[B]
    g = jnp.arange(G_POOL, dtype=jnp.int32)
    b_of_g = jnp.clip(
        jnp.searchsorted(offs, g, side="right").astype(jnp.int32) - 1,
        0, B - 1)
    rel = g - offs[b_of_g]
    blk_g = jnp.clip(blk0[b_of_g] + rel, 0, NRB - 1)
    valid = (g < total).astype(jnp.int32)
    first = ((rel == 0) & (valid > 0)).astype(jnp.int32)
    return jnp.stack([blk_g, b_of_g, first, valid])


def kernel(x, edge_index, batch_index, mlp1_W1, mlp1_b1, bn1_g, bn1_b,
           mlp1_W2, mlp1_b2, mlp2_W1, mlp2_b1, bn2_g, bn2_b, mlp2_W2,
           mlp2_b2, eps1, eps2, eps3, out1_W, out1_b, out2_W, out2_b,
           out3_W, out3_b, out_W, out_b):
    f32 = jnp.float32
    src = edge_index[0].astype(jnp.int32)
    dst = edge_index[1].astype(jnp.int32)
    epad = EROWS * 128 - E
    src2d = jnp.concatenate([src, jnp.zeros((epad,), jnp.int32)]
                            ).reshape(EROWS, 128)
    dst2d = jnp.concatenate([dst, jnp.full((epad,), N, jnp.int32)]
                            ).reshape(EROWS, 128)
    srcoff = (src2d[None] +
              (jnp.arange(NB2, dtype=jnp.int32) * N)[:, None, None])
    srcoff13 = srcoff.reshape(NB2 * EROWS, 128)
    srcoff3 = srcoff[:NB1].reshape(NB1 * EROWS, 128)
    zrows = jnp.zeros((NPT, C), f32)

    x2d = jnp.pad(x.astype(f32), ((0, 0), (0, F1P - x.shape[1])))
    xtab = x2d.reshape(N, NB1, C).transpose(1, 0, 2).reshape(NB1 * N, C)
    w1a = jnp.pad(mlp1_W1, ((0, F1P - mlp1_W1.shape[0]), (0, 0)))
    w1b = jnp.pad(mlp2_W1, ((0, HP - H), (0, 0)))

    def consts(g, b, b2, ob):
        return jnp.concatenate(
            [g[None], b[None], b2[None], ob[None],
             jnp.zeros((4, H), f32)], axis=0)

    c1 = consts(bn1_g, bn1_b, mlp1_b2, out1_b)
    c2 = consts(bn2_g, bn2_b, mlp2_b2, out2_b)
    c3 = consts(bn2_g, bn2_b, mlp2_b2, out3_b)
    fc = (jnp.zeros((8, H), f32)
          .at[0].set(out_W[:H, 0]).at[1].set(out_W[H:, 0])
          .at[2].set(out_b[0]))

    agg = _sc_aggregate(xtab, srcoff3, dst2d, zrows, NB1, F1P)
    z, st = _mm1(x2d, agg, w1a, eps1, F1P)
    t2d, ttab = _tail(z, st, c1, mlp1_W2, out1_W, True)
    agg = _sc_aggregate(ttab.reshape(NB2 * N, C), srcoff13, dst2d, zrows,
                        NB2, HP)
    z, st = _mm1(t2d, agg, w1b, eps2, HP)
    t2d, ttab = _tail(z, st, c2, mlp2_W2, out2_W, True)
    agg = _sc_aggregate(ttab.reshape(NB2 * N, C), srcoff13, dst2d, zrows,
                        NB2, HP)
    z, st = _mm1(t2d, agg, w1b, eps3, HP)
    t3 = _tail(z, st, c3, mlp2_W2, out3_W, False)

    sched = _schedule(batch_index)
    bid8 = jnp.broadcast_to(batch_index.astype(jnp.int32)[:, None], (N, 8))
    mx, sm, ct = _pool(t3, bid8, sched)
    out = _final(mx, sm, ct, fc)
    return out[:, :1]
